# Initial kernel scaffold; baseline (speedup 1.0000x reference)
import jax
import jax.numpy as jnp
from jax.experimental import pallas as pl


def kernel(q, k, v):
    raise NotImplementedError("write your pallas kernel here")


# speedup vs baseline: 9.1488x; 9.1488x over previous
import functools

import jax
import jax.numpy as jnp
import numpy as np
from jax.experimental import pallas as pl
from jax.experimental.pallas import tpu as pltpu

FRAG = 32
G_PER_HEAD = 3
PROTO_COUNT = 16
TOP_U = 8
TOPK_FRAC = 0.2
W_MEAN, W_MAX, W_TOPK, W_STD = 1.0, 0.6, 0.4, 0.2

T_Q = 256
GPAD = 8


def _normalize_safe(x, eps=1e-6):
    n = jnp.linalg.norm(x, axis=-1, keepdims=True)
    return x / jnp.maximum(n, eps)


def _pick_globals(q, k, g_eff):
    B, H, S, D = k.shape
    Kmean = k.mean(axis=0)
    Kbar = _normalize_safe(Kmean)
    q_summary = _normalize_safe(q.mean(axis=0))
    p = min(PROTO_COUNT, S)
    idxp = jnp.round(jnp.linspace(0.0, S - 1, p)).astype(jnp.int32)
    Qp = q_summary[:, idxp, :]
    Smat = jax.nn.relu(jnp.einsum('hkd,hpd->hkp', Kbar, Qp))
    mean = Smat.mean(axis=-1)
    mx = Smat.max(axis=-1)
    kq = max(1, int(round(p * TOPK_FRAC)))
    topk_mean = jax.lax.top_k(Smat, kq)[0].mean(axis=-1)
    std = jnp.std(Smat, axis=-1, ddof=1)
    u_full = W_MEAN * mean + W_MAX * mx + W_TOPK * topk_mean + W_STD * std
    U = min(TOP_U, S)
    U = max(g_eff, min(U, S))
    u_vals, top_idx = jax.lax.top_k(u_full, U)
    S_sub = jnp.take_along_axis(Smat, top_idx[:, :, None], axis=1)
    m = jnp.zeros((H, p), dtype=S_sub.dtype)
    blocked = jnp.zeros((H, U), dtype=bool)
    hr = jnp.arange(H)
    chosen = []
    steps = min(g_eff, U)
    for r in range(steps):
        gains = jax.nn.relu(S_sub - m[:, None, :]).sum(axis=-1)
        gains = jnp.where(blocked, -1e9, gains)
        j = jnp.argmax(gains, axis=1)
        chosen.append(j)
        blocked = blocked.at[hr, j].set(True)
        m = jnp.maximum(m, S_sub[hr, j, :])
    chosen_local = jnp.stack(chosen, axis=1)
    chosen_abs = jnp.take_along_axis(top_idx, chosen_local, axis=1)
    return chosen_abs


def _attn_kernel(g_ref, q_ref, k_ref, v_ref, o_ref, *, S, D, T, L):
    h = pl.program_id(0)
    t = pl.program_id(1)
    t0 = t * T
    base = jnp.clip(t0 - FRAG, 0, S - L)
    scale = 1.0 / np.sqrt(D)

    qb = q_ref[0]
    ks = k_ref[0, pl.ds(base, L), :]
    vs = v_ref[0, pl.ds(base, L), :]

    scores = jax.lax.dot_general(
        qb, ks, (((1,), (1,)), ((), ())),
        preferred_element_type=jnp.float32) * scale

    t_abs = t0 + jax.lax.broadcasted_iota(jnp.int32, (T, L), 0)
    j_abs = base + jax.lax.broadcasted_iota(jnp.int32, (T, L), 1)
    start = jnp.clip(t_abs - FRAG // 2, 0, S - FRAG)
    in_band = (j_abs >= start) & (j_abs < start + FRAG)
    scores = jnp.where(in_band, scores, -1e30)

    rows_k = [k_ref[0, pl.ds(g_ref[h, g], 1), :] for g in range(G_PER_HEAD)]
    rows_v = [v_ref[0, pl.ds(g_ref[h, g], 1), :] for g in range(G_PER_HEAD)]
    pad = jnp.zeros((GPAD - G_PER_HEAD, D), jnp.float32)
    kg = jnp.concatenate(rows_k + [pad], axis=0)
    vg = jnp.concatenate(rows_v + [pad], axis=0)

    gscores = jax.lax.dot_general(
        qb, kg, (((1,), (1,)), ((), ())),
        preferred_element_type=jnp.float32) * scale
    gcol = jax.lax.broadcasted_iota(jnp.int32, (T, GPAD), 1)
    gscores = jnp.where(gcol < G_PER_HEAD, gscores, -1e30)

    m = jnp.maximum(jnp.max(scores, axis=-1, keepdims=True),
                    jnp.max(gscores, axis=-1, keepdims=True))
    pw = jnp.exp(scores - m)
    pg = jnp.exp(gscores - m)
    denom = (jnp.sum(pw, axis=-1, keepdims=True) +
             jnp.sum(pg, axis=-1, keepdims=True))

    out = (jax.lax.dot_general(pw, vs, (((1,), (0,)), ((), ())),
                               preferred_element_type=jnp.float32) +
           jax.lax.dot_general(pg, vg, (((1,), (0,)), ((), ())),
                               preferred_element_type=jnp.float32))
    o_ref[0] = out / denom


def kernel(q, k, v):
    B, H, S, D = q.shape
    g_idx = _pick_globals(q, k, G_PER_HEAD).astype(jnp.int32)

    qh = q.reshape(H, S, D)
    kh = k.reshape(H, S, D)
    vh = v.reshape(H, S, D)

    T = T_Q
    L = T + 2 * FRAG

    grid_spec = pltpu.PrefetchScalarGridSpec(
        num_scalar_prefetch=1,
        grid=(H, S // T),
        in_specs=[
            pl.BlockSpec((1, T, D), lambda h, t, g: (h, t, 0)),
            pl.BlockSpec((1, S, D), lambda h, t, g: (h, 0, 0)),
            pl.BlockSpec((1, S, D), lambda h, t, g: (h, 0, 0)),
        ],
        out_specs=pl.BlockSpec((1, T, D), lambda h, t, g: (h, t, 0)),
    )

    out = pl.pallas_call(
        functools.partial(_attn_kernel, S=S, D=D, T=T, L=L),
        grid_spec=grid_spec,
        out_shape=jax.ShapeDtypeStruct((H, S, D), jnp.float32),
        compiler_params=pltpu.CompilerParams(
            dimension_semantics=("arbitrary", "arbitrary")),
    )(g_idx, qh, kh, vh)

    return out.reshape(B, H, S, D)
